# baseline (speedup 1.0000x reference)
import functools

import jax
import jax.numpy as jnp
from jax.experimental import pallas as pl

E = 64
N_GROUP = 8
TOPK_GROUP = 3
TOP_K = 8
GROUP_SIZE = E // N_GROUP


def _gate_block(x_ref, w_ref, idx_ref, wgt_ref):
    x = x_ref[...]
    w = w_ref[...]
    lt = jax.lax.dot_general(
        w, x, (((1,), (1,)), ((), ())),
        preferred_element_type=jnp.float32)
    tb = lt.shape[1]
    ninf = jnp.float32(-jnp.inf)

    g = jnp.max(lt.reshape(N_GROUP, GROUP_SIZE, tb), axis=1)

    ones_row = jnp.ones((1, E), dtype=jnp.bfloat16)
    ex16 = jnp.exp(lt).astype(jnp.bfloat16)
    den = jax.lax.dot_general(
        ones_row, ex16, (((1,), (0,)), ((), ())),
        preferred_element_type=jnp.float32)
    rden = 1.0 / den

    sel = jnp.zeros((N_GROUP, tb), dtype=jnp.float32)
    work = g
    for _ in range(TOPK_GROUP):
        gm = jnp.max(work, axis=0, keepdims=True)
        eq = work == gm
        sel = sel + jnp.where(eq, 1.0, 0.0)
        work = jnp.where(eq, ninf, work)

    sel64 = jnp.broadcast_to(
        sel.reshape(N_GROUP, 1, tb),
        (N_GROUP, GROUP_SIZE, tb)).reshape(E, tb)
    cand = jnp.where(sel64 > 0.0, lt, ninf)

    iota_row = jax.lax.broadcasted_iota(
        jnp.int32, (1, E), 1).astype(jnp.bfloat16)
    work = cand
    for k in range(TOP_K):
        km = jnp.max(work, axis=0, keepdims=True)
        eq = work == km
        work = jnp.where(eq, ninf, work)
        fidx_f = jax.lax.dot_general(
            iota_row, eq.astype(jnp.bfloat16), (((1,), (0,)), ((), ())),
            preferred_element_type=jnp.float32)
        idx_ref[k:k + 1, :] = fidx_f.astype(jnp.int32)
        wgt_ref[k:k + 1, :] = jnp.exp(km) * rden


@functools.partial(jax.jit, static_argnames=())
def kernel(x, W):
    b, s, h = x.shape
    t = b * s
    xs = x.reshape(t, h)
    tb = 2048
    grid = (t // tb,)
    idx_t, wgt_t = pl.pallas_call(
        _gate_block,
        grid=grid,
        in_specs=[
            pl.BlockSpec((tb, h), lambda i: (i, 0)),
            pl.BlockSpec((E, h), lambda i: (0, 0)),
        ],
        out_specs=[
            pl.BlockSpec((TOP_K, tb), lambda i: (0, i)),
            pl.BlockSpec((TOP_K, tb), lambda i: (0, i)),
        ],
        out_shape=[
            jax.ShapeDtypeStruct((TOP_K, t), jnp.int32),
            jax.ShapeDtypeStruct((TOP_K, t), jnp.float32),
        ],
    )(xs, W)
    return idx_t.T, wgt_t.T


# speedup vs baseline: 1.1129x; 1.1129x over previous
import functools

import jax
import jax.numpy as jnp
from jax.experimental import pallas as pl

E = 64
N_GROUP = 8
TOPK_GROUP = 3
TOP_K = 8
GROUP_SIZE = E // N_GROUP


def _gate_block(x_ref, w_ref, idx_ref, wgt_ref):
    x = x_ref[...]
    w = w_ref[...]
    lt = jax.lax.dot_general(
        w, x, (((1,), (1,)), ((), ())),
        preferred_element_type=jnp.float32)
    tb = lt.shape[1]
    ninf = jnp.float32(-jnp.inf)

    g = jnp.max(lt.reshape(N_GROUP, GROUP_SIZE, tb), axis=1)

    ones_row = jnp.ones((1, E), dtype=jnp.bfloat16)
    ex16 = jnp.exp(lt).astype(jnp.bfloat16)
    den = jax.lax.dot_general(
        ones_row, ex16, (((1,), (0,)), ((), ())),
        preferred_element_type=jnp.float32)
    rden = 1.0 / den

    sel = jnp.zeros((N_GROUP, tb), dtype=jnp.float32)
    work = g
    for _ in range(TOPK_GROUP):
        gm = jnp.max(work, axis=0, keepdims=True)
        eq = work == gm
        sel = sel + jnp.where(eq, 1.0, 0.0)
        work = jnp.where(eq, ninf, work)

    sel64 = jnp.broadcast_to(
        sel.reshape(N_GROUP, 1, tb),
        (N_GROUP, GROUP_SIZE, tb)).reshape(E, tb)
    cand = jnp.where(sel64 > 0.0, lt, ninf)

    iota_row = jax.lax.broadcasted_iota(
        jnp.int32, (1, E), 1).astype(jnp.bfloat16)
    work = cand
    for k in range(TOP_K):
        km = jnp.max(work, axis=0, keepdims=True)
        eq = work == km
        work = jnp.where(eq, ninf, work)
        fidx_f = jax.lax.dot_general(
            iota_row, eq.astype(jnp.bfloat16), (((1,), (0,)), ((), ())),
            preferred_element_type=jnp.float32)
        idx_ref[k:k + 1, :] = fidx_f.astype(jnp.int32)
        wgt_ref[k:k + 1, :] = jnp.exp(km) * rden


@functools.partial(jax.jit, static_argnames=())
def kernel(x, W):
    b, s, h = x.shape
    t = b * s
    xs = x.reshape(t, h)
    tb = 4096
    grid = (t // tb,)
    idx_t, wgt_t = pl.pallas_call(
        _gate_block,
        grid=grid,
        in_specs=[
            pl.BlockSpec((tb, h), lambda i: (i, 0)),
            pl.BlockSpec((E, h), lambda i: (0, 0)),
        ],
        out_specs=[
            pl.BlockSpec((TOP_K, tb), lambda i: (0, i)),
            pl.BlockSpec((TOP_K, tb), lambda i: (0, i)),
        ],
        out_shape=[
            jax.ShapeDtypeStruct((TOP_K, t), jnp.int32),
            jax.ShapeDtypeStruct((TOP_K, t), jnp.float32),
        ],
    )(xs, W)
    return idx_t.T, wgt_t.T
